# baseline (speedup 1.0000x reference)
import functools
import jax
import jax.numpy as jnp
from jax import lax
from jax.experimental import pallas as pl
from jax.experimental.pallas import tpu as pltpu
from jax.experimental.pallas import tpu_sc as plsc

_NC = 2
_NS = 16
_NW = _NC * _NS


def kernel(src, table):
    seq_len, batch = src.shape
    max_len, hidden = table.shape

    rows_per_w = seq_len // _NW
    sizes = [56, 56, 56, 56, 32]
    offs = [0, 56, 112, 168, 224]
    n_chunks = len(sizes)
    nbuf = 2
    bufrows = max(sizes)

    mesh = plsc.VectorSubcoreMesh(core_axis_name="c", subcore_axis_name="s")

    @functools.partial(
        pl.kernel,
        mesh=mesh,
        out_type=jax.ShapeDtypeStruct((seq_len, batch, hidden), jnp.float32),
        scratch_types=[
            [pltpu.VMEM((bufrows, hidden), jnp.float32) for _ in range(nbuf)],
            pltpu.SemaphoreType.DMA,
            [pltpu.SemaphoreType.DMA for _ in range(nbuf)],
        ],
    )
    def k(table_hbm, out_hbm, bufs, rsem, wsems):
        c = lax.axis_index("c")
        s = lax.axis_index("s")
        wid = s * _NC + c
        base = wid * rows_per_w

        def read(j):
            r0 = base + offs[j]
            return pltpu.async_copy(
                table_hbm.at[pl.ds(r0, sizes[j])],
                bufs[j % nbuf].at[pl.ds(0, sizes[j])],
                rsem,
            )

        def write(j):
            r0 = base + offs[j]
            return [
                pltpu.async_copy(
                    bufs[j % nbuf].at[pl.ds(0, sizes[j])],
                    out_hbm.at[pl.ds(r0, sizes[j]), b],
                    wsems[j % nbuf],
                )
                for b in range(batch)
            ]

        writes = [None] * n_chunks
        reads = [read(0)]
        for j in range(n_chunks):
            reads[j].wait()
            if j == 0:
                @pl.when(wid == 0)
                def _():
                    def zb(i, c2):
                        bufs[0][0, pl.ds(i * 16, 16)] = jnp.zeros(
                            (16,), jnp.float32
                        )
                        return c2
                    lax.fori_loop(0, hidden // 16, zb, 0)
            writes[j] = write(j)
            if j + 1 < n_chunks:
                if j - (nbuf - 1) >= 0:
                    for w in writes[j - (nbuf - 1)]:
                        w.wait()
                reads.append(read(j + 1))

        for j in range(max(0, n_chunks - nbuf), n_chunks):
            for w in writes[j]:
                w.wait()

    return k(table)


# speedup vs baseline: 1.0710x; 1.0710x over previous
import functools
import jax
import jax.numpy as jnp
from jax import lax
from jax.experimental import pallas as pl
from jax.experimental.pallas import tpu as pltpu
from jax.experimental.pallas import tpu_sc as plsc

_NC = 2
_NS = 16
_NW = _NC * _NS


def kernel(src, table):
    seq_len, batch = src.shape
    max_len, hidden = table.shape

    rows_per_w = seq_len // _NW
    sizes = [56, 56, 56, 56, 32]
    offs = [0, 56, 112, 168, 224]
    n_chunks = len(sizes)
    nbuf = 2
    bufrows = max(sizes)

    mesh = plsc.VectorSubcoreMesh(core_axis_name="c", subcore_axis_name="s")

    @functools.partial(
        pl.kernel,
        mesh=mesh,
        out_type=jax.ShapeDtypeStruct((seq_len, batch, hidden), jnp.float32),
        scratch_types=[
            [pltpu.VMEM((bufrows, hidden), jnp.float32) for _ in range(nbuf)],
            pltpu.SemaphoreType.DMA,
            [pltpu.SemaphoreType.DMA for _ in range(nbuf)],
        ],
    )
    def k(table_hbm, out_hbm, bufs, rsem, wsems):
        c = lax.axis_index("c")
        s = lax.axis_index("s")
        wid = s * _NC + c
        base = wid * rows_per_w

        def read(j):
            r0 = base + offs[j]
            return pltpu.async_copy(
                table_hbm.at[pl.ds(r0, sizes[j])],
                bufs[j % nbuf].at[pl.ds(0, sizes[j])],
                rsem,
            )

        def write(j):
            r0 = base + offs[j]
            return [
                pltpu.async_copy(
                    bufs[j % nbuf].at[pl.ds(0, sizes[j])],
                    out_hbm.at[pl.ds(r0, sizes[j]), b],
                    wsems[j % nbuf],
                )
                for b in range(batch)
            ]

        writes = [None] * n_chunks
        reads = [read(0)]
        for j in range(n_chunks):
            reads[j].wait()
            if j == 0:
                @pl.when(wid == 0)
                def _():
                    def zb(i, c2):
                        bufs[0][0, pl.ds(i * 16, 16)] = jnp.zeros(
                            (16,), jnp.float32
                        )
                        return c2
                    lax.fori_loop(0, hidden // 16, zb, 0)
            if j + 1 < n_chunks:
                if j - (nbuf - 1) >= 0:
                    for w in writes[j - (nbuf - 1)]:
                        w.wait()
                reads.append(read(j + 1))
            writes[j] = write(j)

        for j in range(max(0, n_chunks - nbuf), n_chunks):
            for w in writes[j]:
                w.wait()

    return k(table)
